# Initial kernel scaffold; baseline (speedup 1.0000x reference)
import jax
import jax.numpy as jnp
from jax.experimental import pallas as pl


def kernel(hidden_states, router_weight, w_gate, w_up, w_down):
    raise NotImplementedError("write your pallas kernel here")


# speedup vs baseline: 2.5391x; 2.5391x over previous
import functools

import jax
import jax.numpy as jnp
from jax import lax
from jax.experimental import pallas as pl
from jax.experimental.pallas import tpu as pltpu
from jax.experimental.pallas import tpu_sc as plsc

E = 64
K = 2
D = 2048
FF = 768
T = 2048
TB = 128
TM = 128
NT = T * K // TM + E
R_PAD = NT * TM

NW = 32
RPW = R_PAD // NW
CH = 48
TPW = T // NW
CT = 16


def _router_body(x_ref, rw_ref, e0_ref, e1_ref, w0_ref, w1_ref,
                 r0_ref, r1_ref, cnt_ref, carry_ref):
    i = pl.program_id(0)

    @pl.when(i == 0)
    def _init():
        carry_ref[...] = jnp.zeros_like(carry_ref)

    logits = jnp.dot(x_ref[...], rw_ref[...],
                     preferred_element_type=jnp.float32)
    lane = lax.broadcasted_iota(jnp.int32, logits.shape, 1)
    m0 = jnp.max(logits, axis=1, keepdims=True)
    c0 = jnp.min(jnp.where(logits == m0, lane, E), axis=1, keepdims=True)
    oh0 = lane == c0
    masked = jnp.where(oh0, -jnp.inf, logits)
    m1 = jnp.max(masked, axis=1, keepdims=True)
    c1 = jnp.min(jnp.where(masked == m1, lane, E), axis=1, keepdims=True)
    oh1 = lane == c1
    w0 = 1.0 / (1.0 + jnp.exp(m1 - m0))
    w1 = 1.0 - w0

    oh = jnp.where(oh0 | oh1, 1.0, 0.0)
    row = lax.broadcasted_iota(jnp.int32, (TB, TB), 0)
    col = lax.broadcasted_iota(jnp.int32, (TB, TB), 1)
    tri = jnp.where(col < row, 1.0, 0.0)
    cb = jnp.dot(tri, oh, preferred_element_type=jnp.float32)
    rank = cb + carry_ref[0:1, :]
    r0 = jnp.sum(jnp.where(oh0, rank, 0.0), axis=1)
    r1 = jnp.sum(jnp.where(oh1, rank, 0.0), axis=1)
    carry_ref[...] = carry_ref[...] + jnp.sum(oh, axis=0, keepdims=True)
    cnt_ref[...] = carry_ref[...].astype(jnp.int32)

    e0_ref[0, 0, :] = c0[:, 0]
    e1_ref[0, 0, :] = c1[:, 0]
    w0_ref[0, 0, :] = w0[:, 0]
    w1_ref[0, 0, :] = w1[:, 0]
    r0_ref[0, 0, :] = r0.astype(jnp.int32)
    r1_ref[0, 0, :] = r1.astype(jnp.int32)


def _router(x, rw):
    nblk = T // TB
    i32 = jnp.int32
    f32 = jnp.float32
    outs = pl.pallas_call(
        _router_body,
        grid=(nblk,),
        in_specs=[
            pl.BlockSpec((TB, D), lambda i: (i, 0)),
            pl.BlockSpec((D, E), lambda i: (0, 0)),
        ],
        out_specs=[
            pl.BlockSpec((1, 1, TB), lambda i: (i, 0, 0)),
            pl.BlockSpec((1, 1, TB), lambda i: (i, 0, 0)),
            pl.BlockSpec((1, 1, TB), lambda i: (i, 0, 0)),
            pl.BlockSpec((1, 1, TB), lambda i: (i, 0, 0)),
            pl.BlockSpec((1, 1, TB), lambda i: (i, 0, 0)),
            pl.BlockSpec((1, 1, TB), lambda i: (i, 0, 0)),
            pl.BlockSpec((8, E), lambda i: (0, 0)),
        ],
        out_shape=[
            jax.ShapeDtypeStruct((nblk, 1, TB), i32),
            jax.ShapeDtypeStruct((nblk, 1, TB), i32),
            jax.ShapeDtypeStruct((nblk, 1, TB), f32),
            jax.ShapeDtypeStruct((nblk, 1, TB), f32),
            jax.ShapeDtypeStruct((nblk, 1, TB), i32),
            jax.ShapeDtypeStruct((nblk, 1, TB), i32),
            jax.ShapeDtypeStruct((8, E), i32),
        ],
        scratch_shapes=[pltpu.VMEM((8, E), f32)],
    )(x, rw)
    e0, e1, w0, w1, r0, r1, cnt = outs
    flat = lambda a: a.reshape(T)
    return (flat(e0), flat(e1), flat(w0), flat(w1), flat(r0), flat(r1),
            cnt[0])


def _sc_dispatch(x, token_map):
    mesh = plsc.VectorSubcoreMesh(core_axis_name="c", subcore_axis_name="s")

    @functools.partial(
        pl.kernel,
        out_type=jax.ShapeDtypeStruct((R_PAD, D), jnp.float32),
        mesh=mesh,
        scratch_types=[
            pltpu.VMEM((RPW,), jnp.int32),
            pltpu.VMEM((CH, D), jnp.float32),
            pltpu.SemaphoreType.DMA,
        ],
    )
    def k(x_hbm, tok_hbm, xs_hbm, tok_v, rows_v, sem):
        wid = lax.axis_index("s") * 2 + lax.axis_index("c")
        base = wid * RPW
        pltpu.sync_copy(tok_hbm.at[pl.ds(base, RPW)], tok_v)
        for c in range(RPW // CH):
            idx = tok_v.at[pl.ds(c * CH, CH)]
            pltpu.async_copy(x_hbm.at[idx], rows_v, sem).wait()
            pltpu.sync_copy(rows_v, xs_hbm.at[pl.ds(base + c * CH, CH)])

    return k(x, token_map)


def _ffn_body(te_ref, xb_ref, x_ref, wg_ref, wu_ref, wd_ref, y_ref):
    x = x_ref[...]
    g = jnp.dot(x, wg_ref[0], preferred_element_type=jnp.float32)
    u = jnp.dot(x, wu_ref[0], preferred_element_type=jnp.float32)
    h = (g * jax.nn.sigmoid(g)) * u
    y = jnp.dot(h, wd_ref[0], preferred_element_type=jnp.float32)
    y_ref[...] = y


def _ffn(xs, wg, wu, wd, tile_expert, x_blk):
    grid_spec = pltpu.PrefetchScalarGridSpec(
        num_scalar_prefetch=2,
        grid=(NT,),
        in_specs=[
            pl.BlockSpec((TM, D), lambda i, te, xb: (xb[i], 0)),
            pl.BlockSpec((1, D, FF), lambda i, te, xb: (te[i], 0, 0)),
            pl.BlockSpec((1, D, FF), lambda i, te, xb: (te[i], 0, 0)),
            pl.BlockSpec((1, FF, D), lambda i, te, xb: (te[i], 0, 0)),
        ],
        out_specs=pl.BlockSpec((TM, D), lambda i, te, xb: (xb[i], 0)),
    )
    return pl.pallas_call(
        _ffn_body,
        grid_spec=grid_spec,
        out_shape=jax.ShapeDtypeStruct((R_PAD, D), jnp.float32),
    )(tile_expert, x_blk, xs, wg, wu, wd)


def _sc_combine(y, d0, d1, w0b, w1b):
    mesh = plsc.VectorSubcoreMesh(core_axis_name="c", subcore_axis_name="s")

    @functools.partial(
        pl.kernel,
        out_type=jax.ShapeDtypeStruct((T, D), jnp.float32),
        mesh=mesh,
        scratch_types=[
            pltpu.VMEM((TPW,), jnp.int32),
            pltpu.VMEM((TPW,), jnp.int32),
            pltpu.VMEM((TPW, 16), jnp.float32),
            pltpu.VMEM((TPW, 16), jnp.float32),
            pltpu.VMEM((CT, D), jnp.float32),
            pltpu.VMEM((CT, D), jnp.float32),
            pltpu.SemaphoreType.DMA,
            pltpu.SemaphoreType.DMA,
        ],
    )
    def k(y_hbm, d0_hbm, d1_hbm, w0_hbm, w1_hbm, out_hbm,
          d0_v, d1_v, w0_v, w1_v, r0_v, r1_v, s0, s1):
        wid = lax.axis_index("s") * 2 + lax.axis_index("c")
        base = wid * TPW
        pltpu.sync_copy(d0_hbm.at[pl.ds(base, TPW)], d0_v)
        pltpu.sync_copy(d1_hbm.at[pl.ds(base, TPW)], d1_v)
        pltpu.sync_copy(w0_hbm.at[pl.ds(base, TPW)], w0_v)
        pltpu.sync_copy(w1_hbm.at[pl.ds(base, TPW)], w1_v)
        for c in range(TPW // CT):
            i0 = d0_v.at[pl.ds(c * CT, CT)]
            i1 = d1_v.at[pl.ds(c * CT, CT)]
            cp0 = pltpu.async_copy(y_hbm.at[i0], r0_v, s0)
            cp1 = pltpu.async_copy(y_hbm.at[i1], r1_v, s1)
            cp0.wait()
            cp1.wait()

            def add_row(i, _):
                b0 = w0_v[c * CT + i, :]
                b1 = w1_v[c * CT + i, :]

                def add_vec(j, _):
                    r0_v[i, pl.ds(j * 16, 16)] = (
                        b0 * r0_v[i, pl.ds(j * 16, 16)]
                        + b1 * r1_v[i, pl.ds(j * 16, 16)])
                    return 0
                lax.fori_loop(0, D // 16, add_vec, 0)
                return 0

            lax.fori_loop(0, CT, add_row, 0)
            pltpu.sync_copy(r0_v, out_hbm.at[pl.ds(base + c * CT, CT)])

    return k(y, d0, d1, w0b, w1b)


def kernel(hidden_states, router_weight, w_gate, w_up, w_down):
    x = hidden_states
    i32 = jnp.int32

    e0, e1, w0, w1, r0, r1, cnt = _router(x, router_weight)
    sizes = cnt[:E]

    nt = (sizes + TM - 1) // TM
    starts = (jnp.cumsum(nt) - nt) * TM
    dest0 = (starts[e0] + r0).astype(i32)
    dest1 = (starts[e1] + r1).astype(i32)
    slot = jnp.arange(T, dtype=i32)
    token_map = (jnp.zeros((R_PAD,), i32)
                 .at[dest0].set(slot).at[dest1].set(slot))

    nt_cum = jnp.cumsum(nt)
    total_tiles = nt_cum[E - 1]
    t_ar = jnp.arange(NT, dtype=i32)
    te = jnp.searchsorted(nt_cum, t_ar, side="right").astype(i32)
    last_e = te[jnp.maximum(total_tiles - 1, 0)]
    tile_expert = jnp.where(t_ar < total_tiles, te, last_e).astype(i32)
    x_blk = jnp.where(t_ar < total_tiles, t_ar, total_tiles - 1).astype(i32)

    xs = _sc_dispatch(x, token_map)
    y = _ffn(xs, w_gate, w_up, w_down, tile_expert, x_blk)
    w0b = jnp.broadcast_to(w0[:, None], (T, 16))
    w1b = jnp.broadcast_to(w1[:, None], (T, 16))
    return _sc_combine(y, dest0, dest1, w0b, w1b)


# speedup vs baseline: 3.2555x; 1.2822x over previous
import functools

import jax
import jax.numpy as jnp
from jax import lax
from jax.experimental import pallas as pl
from jax.experimental.pallas import tpu as pltpu
from jax.experimental.pallas import tpu_sc as plsc

E = 64
K = 2
D = 2048
FF = 768
T = 2048
TB = 128
TM = 128
NT = T * K // TM + E
R_PAD = NT * TM

NW = 32
RPW = R_PAD // NW
CH = 48
NCH = R_PAD // CH
NCPW = NCH // NW
TPW = T // NW
CT = 16


def _router_body(x_ref, rw_ref, e0_ref, e1_ref, w0_ref, w1_ref,
                 r0_ref, r1_ref, cnt_ref, carry_ref):
    i = pl.program_id(0)

    @pl.when(i == 0)
    def _init():
        carry_ref[...] = jnp.zeros_like(carry_ref)

    logits = jnp.dot(x_ref[...], rw_ref[...],
                     preferred_element_type=jnp.float32)
    lane = lax.broadcasted_iota(jnp.int32, logits.shape, 1)
    m0 = jnp.max(logits, axis=1, keepdims=True)
    c0 = jnp.min(jnp.where(logits == m0, lane, E), axis=1, keepdims=True)
    oh0 = lane == c0
    masked = jnp.where(oh0, -jnp.inf, logits)
    m1 = jnp.max(masked, axis=1, keepdims=True)
    c1 = jnp.min(jnp.where(masked == m1, lane, E), axis=1, keepdims=True)
    oh1 = lane == c1
    w0 = 1.0 / (1.0 + jnp.exp(m1 - m0))
    w1 = 1.0 - w0

    oh = jnp.where(oh0 | oh1, 1.0, 0.0)
    row = lax.broadcasted_iota(jnp.int32, (TB, TB), 0)
    col = lax.broadcasted_iota(jnp.int32, (TB, TB), 1)
    tri = jnp.where(col < row, 1.0, 0.0)
    cb = jnp.dot(tri, oh, preferred_element_type=jnp.float32)
    rank = cb + carry_ref[0:1, :]
    r0 = jnp.sum(jnp.where(oh0, rank, 0.0), axis=1)
    r1 = jnp.sum(jnp.where(oh1, rank, 0.0), axis=1)
    carry_ref[...] = carry_ref[...] + jnp.sum(oh, axis=0, keepdims=True)
    cnt_ref[...] = carry_ref[...].astype(jnp.int32)

    e0_ref[0, 0, :] = c0[:, 0]
    e1_ref[0, 0, :] = c1[:, 0]
    w0_ref[0, 0, :] = w0[:, 0]
    w1_ref[0, 0, :] = w1[:, 0]
    r0_ref[0, 0, :] = r0.astype(jnp.int32)
    r1_ref[0, 0, :] = r1.astype(jnp.int32)


def _router(x, rw):
    nblk = T // TB
    i32 = jnp.int32
    f32 = jnp.float32
    outs = pl.pallas_call(
        _router_body,
        grid=(nblk,),
        in_specs=[
            pl.BlockSpec((TB, D), lambda i: (i, 0)),
            pl.BlockSpec((D, E), lambda i: (0, 0)),
        ],
        out_specs=[
            pl.BlockSpec((1, 1, TB), lambda i: (i, 0, 0)),
            pl.BlockSpec((1, 1, TB), lambda i: (i, 0, 0)),
            pl.BlockSpec((1, 1, TB), lambda i: (i, 0, 0)),
            pl.BlockSpec((1, 1, TB), lambda i: (i, 0, 0)),
            pl.BlockSpec((1, 1, TB), lambda i: (i, 0, 0)),
            pl.BlockSpec((1, 1, TB), lambda i: (i, 0, 0)),
            pl.BlockSpec((8, E), lambda i: (0, 0)),
        ],
        out_shape=[
            jax.ShapeDtypeStruct((nblk, 1, TB), i32),
            jax.ShapeDtypeStruct((nblk, 1, TB), i32),
            jax.ShapeDtypeStruct((nblk, 1, TB), f32),
            jax.ShapeDtypeStruct((nblk, 1, TB), f32),
            jax.ShapeDtypeStruct((nblk, 1, TB), i32),
            jax.ShapeDtypeStruct((nblk, 1, TB), i32),
            jax.ShapeDtypeStruct((8, E), i32),
        ],
        scratch_shapes=[pltpu.VMEM((8, E), f32)],
    )(x, rw)
    e0, e1, w0, w1, r0, r1, cnt = outs
    flat = lambda a: a.reshape(T)
    return (flat(e0), flat(e1), flat(w0), flat(w1), flat(r0), flat(r1),
            cnt[0])


def _sc_dispatch(x, token_map, nchunks):
    mesh = plsc.VectorSubcoreMesh(core_axis_name="c", subcore_axis_name="s")

    @functools.partial(
        pl.kernel,
        out_type=jax.ShapeDtypeStruct((R_PAD, D), jnp.float32),
        mesh=mesh,
        scratch_types=[
            pltpu.VMEM((16,), jnp.int32),
            pltpu.VMEM((CH,), jnp.int32),
            pltpu.VMEM((CH, D), jnp.float32),
            pltpu.SemaphoreType.DMA,
        ],
    )
    def k(x_hbm, tok_hbm, nc_hbm, xs_hbm, nc_v, tok_v, rows_v, sem):
        wid = lax.axis_index("s") * 2 + lax.axis_index("c")
        pltpu.sync_copy(nc_hbm, nc_v)
        nc = nc_v[...][0]
        for j in range(NCPW):
            c = wid + j * NW

            @pl.when(c < nc)
            def _():
                pltpu.sync_copy(tok_hbm.at[pl.ds(c * CH, CH)], tok_v)
                pltpu.async_copy(x_hbm.at[tok_v], rows_v, sem).wait()
                pltpu.sync_copy(rows_v, xs_hbm.at[pl.ds(c * CH, CH)])

    return k(x, token_map, nchunks)


def _ffn_body(te_ref, xb_ref, vld_ref, x_ref, wg_ref, wu_ref, wd_ref, y_ref):
    i = pl.program_id(0)

    @pl.when(vld_ref[i] == 1)
    def _():
        x = x_ref[...]
        g = jnp.dot(x, wg_ref[0], preferred_element_type=jnp.float32)
        u = jnp.dot(x, wu_ref[0], preferred_element_type=jnp.float32)
        h = (g * jax.nn.sigmoid(g)) * u
        y = jnp.dot(h, wd_ref[0], preferred_element_type=jnp.float32)
        y_ref[...] = y


def _ffn(xs, wg, wu, wd, tile_expert, x_blk, tile_valid):
    grid_spec = pltpu.PrefetchScalarGridSpec(
        num_scalar_prefetch=3,
        grid=(NT,),
        in_specs=[
            pl.BlockSpec((TM, D), lambda i, te, xb, vl: (xb[i], 0)),
            pl.BlockSpec((1, D, FF), lambda i, te, xb, vl: (te[i], 0, 0)),
            pl.BlockSpec((1, D, FF), lambda i, te, xb, vl: (te[i], 0, 0)),
            pl.BlockSpec((1, FF, D), lambda i, te, xb, vl: (te[i], 0, 0)),
        ],
        out_specs=pl.BlockSpec((TM, D), lambda i, te, xb, vl: (xb[i], 0)),
    )
    return pl.pallas_call(
        _ffn_body,
        grid_spec=grid_spec,
        out_shape=jax.ShapeDtypeStruct((R_PAD, D), jnp.float32),
    )(tile_expert, x_blk, tile_valid, xs, wg, wu, wd)


def _sc_combine(y, d0, d1, w0b, w1b):
    mesh = plsc.VectorSubcoreMesh(core_axis_name="c", subcore_axis_name="s")

    @functools.partial(
        pl.kernel,
        out_type=jax.ShapeDtypeStruct((T, D), jnp.float32),
        mesh=mesh,
        scratch_types=[
            pltpu.VMEM((TPW,), jnp.int32),
            pltpu.VMEM((TPW,), jnp.int32),
            pltpu.VMEM((TPW, 16), jnp.float32),
            pltpu.VMEM((TPW, 16), jnp.float32),
            pltpu.VMEM((CT, D), jnp.float32),
            pltpu.VMEM((CT, D), jnp.float32),
            pltpu.SemaphoreType.DMA,
            pltpu.SemaphoreType.DMA,
        ],
    )
    def k(y_hbm, d0_hbm, d1_hbm, w0_hbm, w1_hbm, out_hbm,
          d0_v, d1_v, w0_v, w1_v, r0_v, r1_v, s0, s1):
        wid = lax.axis_index("s") * 2 + lax.axis_index("c")
        base = wid * TPW
        pltpu.sync_copy(d0_hbm.at[pl.ds(base, TPW)], d0_v)
        pltpu.sync_copy(d1_hbm.at[pl.ds(base, TPW)], d1_v)
        pltpu.sync_copy(w0_hbm.at[pl.ds(base, TPW)], w0_v)
        pltpu.sync_copy(w1_hbm.at[pl.ds(base, TPW)], w1_v)
        for c in range(TPW // CT):
            i0 = d0_v.at[pl.ds(c * CT, CT)]
            i1 = d1_v.at[pl.ds(c * CT, CT)]
            cp0 = pltpu.async_copy(y_hbm.at[i0], r0_v, s0)
            cp1 = pltpu.async_copy(y_hbm.at[i1], r1_v, s1)
            cp0.wait()
            cp1.wait()

            def add_row(i, _):
                b0 = w0_v[c * CT + i, :]
                b1 = w1_v[c * CT + i, :]

                def add_vec(j, _):
                    r0_v[i, pl.ds(j * 16, 16)] = (
                        b0 * r0_v[i, pl.ds(j * 16, 16)]
                        + b1 * r1_v[i, pl.ds(j * 16, 16)])
                    return 0
                lax.fori_loop(0, D // 16, add_vec, 0)
                return 0

            lax.fori_loop(0, CT, add_row, 0)
            pltpu.sync_copy(r0_v, out_hbm.at[pl.ds(base + c * CT, CT)])

    return k(y, d0, d1, w0b, w1b)


def kernel(hidden_states, router_weight, w_gate, w_up, w_down):
    x = hidden_states
    i32 = jnp.int32

    e0, e1, w0, w1, r0, r1, cnt = _router(x, router_weight)
    sizes = cnt[:E]

    nt = (sizes + TM - 1) // TM
    starts = (jnp.cumsum(nt) - nt) * TM
    dest0 = (starts[e0] + r0).astype(i32)
    dest1 = (starts[e1] + r1).astype(i32)
    slot = jnp.arange(T, dtype=i32)
    token_map = (jnp.zeros((R_PAD,), i32)
                 .at[dest0].set(slot).at[dest1].set(slot))

    nt_cum = jnp.cumsum(nt)
    total_tiles = nt_cum[E - 1]
    t_ar = jnp.arange(NT, dtype=i32)
    te = jnp.searchsorted(nt_cum, t_ar, side="right").astype(i32)
    last_e = te[jnp.maximum(total_tiles - 1, 0)]
    tile_valid = (t_ar < total_tiles).astype(i32)
    tile_expert = jnp.where(t_ar < total_tiles, te, last_e).astype(i32)
    x_blk = jnp.where(t_ar < total_tiles, t_ar, total_tiles - 1).astype(i32)

    nch_dyn = (total_tiles * TM + CH - 1) // CH
    nchunks = jnp.full((16,), nch_dyn, i32)

    xs = _sc_dispatch(x, token_map, nchunks)
    y = _ffn(xs, w_gate, w_up, w_down, tile_expert, x_blk, tile_valid)
    w0b = jnp.broadcast_to(w0[:, None], (T, 16))
    w1b = jnp.broadcast_to(w1[:, None], (T, 16))
    return _sc_combine(y, dest0, dest1, w0b, w1b)


# speedup vs baseline: 4.7537x; 1.4602x over previous
import functools

import jax
import jax.numpy as jnp
from jax import lax
from jax.experimental import pallas as pl
from jax.experimental.pallas import tpu as pltpu
from jax.experimental.pallas import tpu_sc as plsc

E = 64
K = 2
D = 2048
FF = 768
T = 2048
TB = 128
TM = 128
NT = T * K // TM + E
R_PAD = NT * TM

NW = 32
RPW = R_PAD // NW
CH = 48
NCH = R_PAD // CH
NCPW = NCH // NW
TPW = T // NW
CT = 16


def _router_body(x_ref, rw_ref, e0_ref, e1_ref, w0_ref, w1_ref,
                 r0_ref, r1_ref, cnt_ref, carry_ref):
    i = pl.program_id(0)

    @pl.when(i == 0)
    def _init():
        carry_ref[...] = jnp.zeros_like(carry_ref)

    logits = jnp.dot(x_ref[...], rw_ref[...],
                     preferred_element_type=jnp.float32)
    lane = lax.broadcasted_iota(jnp.int32, logits.shape, 1)
    m0 = jnp.max(logits, axis=1, keepdims=True)
    c0 = jnp.min(jnp.where(logits == m0, lane, E), axis=1, keepdims=True)
    oh0 = lane == c0
    masked = jnp.where(oh0, -jnp.inf, logits)
    m1 = jnp.max(masked, axis=1, keepdims=True)
    c1 = jnp.min(jnp.where(masked == m1, lane, E), axis=1, keepdims=True)
    oh1 = lane == c1
    w0 = 1.0 / (1.0 + jnp.exp(m1 - m0))
    w1 = 1.0 - w0

    oh = jnp.where(oh0 | oh1, 1.0, 0.0)
    row = lax.broadcasted_iota(jnp.int32, (TB, TB), 0)
    col = lax.broadcasted_iota(jnp.int32, (TB, TB), 1)
    tri = jnp.where(col < row, 1.0, 0.0)
    cb = jnp.dot(tri, oh, preferred_element_type=jnp.float32)
    rank = cb + carry_ref[0:1, :]
    r0 = jnp.sum(jnp.where(oh0, rank, 0.0), axis=1)
    r1 = jnp.sum(jnp.where(oh1, rank, 0.0), axis=1)
    carry_ref[...] = carry_ref[...] + jnp.sum(oh, axis=0, keepdims=True)
    cnt_ref[...] = carry_ref[...].astype(jnp.int32)

    e0_ref[0, 0, :] = c0[:, 0]
    e1_ref[0, 0, :] = c1[:, 0]
    w0_ref[0, 0, :] = w0[:, 0]
    w1_ref[0, 0, :] = w1[:, 0]
    r0_ref[0, 0, :] = r0.astype(jnp.int32)
    r1_ref[0, 0, :] = r1.astype(jnp.int32)


def _router(x, rw):
    nblk = T // TB
    i32 = jnp.int32
    f32 = jnp.float32
    outs = pl.pallas_call(
        _router_body,
        grid=(nblk,),
        in_specs=[
            pl.BlockSpec((TB, D), lambda i: (i, 0)),
            pl.BlockSpec((D, E), lambda i: (0, 0)),
        ],
        out_specs=[
            pl.BlockSpec((1, 1, TB), lambda i: (i, 0, 0)),
            pl.BlockSpec((1, 1, TB), lambda i: (i, 0, 0)),
            pl.BlockSpec((1, 1, TB), lambda i: (i, 0, 0)),
            pl.BlockSpec((1, 1, TB), lambda i: (i, 0, 0)),
            pl.BlockSpec((1, 1, TB), lambda i: (i, 0, 0)),
            pl.BlockSpec((1, 1, TB), lambda i: (i, 0, 0)),
            pl.BlockSpec((8, E), lambda i: (0, 0)),
        ],
        out_shape=[
            jax.ShapeDtypeStruct((nblk, 1, TB), i32),
            jax.ShapeDtypeStruct((nblk, 1, TB), i32),
            jax.ShapeDtypeStruct((nblk, 1, TB), f32),
            jax.ShapeDtypeStruct((nblk, 1, TB), f32),
            jax.ShapeDtypeStruct((nblk, 1, TB), i32),
            jax.ShapeDtypeStruct((nblk, 1, TB), i32),
            jax.ShapeDtypeStruct((8, E), i32),
        ],
        scratch_shapes=[pltpu.VMEM((8, E), f32)],
    )(x, rw)
    e0, e1, w0, w1, r0, r1, cnt = outs
    flat = lambda a: a.reshape(T)
    return (flat(e0), flat(e1), flat(w0), flat(w1), flat(r0), flat(r1),
            cnt[0])


DCH = 32


def _sc_dispatch(x, d0, d1):
    mesh = plsc.VectorSubcoreMesh(core_axis_name="c", subcore_axis_name="s")

    @functools.partial(
        pl.kernel,
        out_type=jax.ShapeDtypeStruct((R_PAD, D), jnp.float32),
        mesh=mesh,
        scratch_types=[
            pltpu.VMEM((DCH, D), jnp.float32),
            pltpu.VMEM((DCH,), jnp.int32),
            pltpu.VMEM((DCH,), jnp.int32),
            pltpu.SemaphoreType.DMA,
            pltpu.SemaphoreType.DMA,
        ],
    )
    def k(x_hbm, d0_hbm, d1_hbm, xs_hbm, xv, i0_v, i1_v, s0, s1):
        wid = lax.axis_index("s") * 2 + lax.axis_index("c")
        for c in range(T // NW // DCH):
            base = wid * (T // NW) + c * DCH
            pltpu.sync_copy(x_hbm.at[pl.ds(base, DCH)], xv)
            pltpu.sync_copy(d0_hbm.at[pl.ds(base, DCH)], i0_v)
            pltpu.sync_copy(d1_hbm.at[pl.ds(base, DCH)], i1_v)
            cp0 = pltpu.async_copy(xv, xs_hbm.at[i0_v], s0)
            cp1 = pltpu.async_copy(xv, xs_hbm.at[i1_v], s1)
            cp0.wait()
            cp1.wait()

    return k(x, d0, d1)


def _ffn_body(te_ref, xb_ref, vld_ref, x_ref, wg_ref, wu_ref, wd_ref, y_ref):
    i = pl.program_id(0)

    @pl.when(vld_ref[i] == 1)
    def _():
        x = x_ref[...]
        g = jnp.dot(x, wg_ref[0], preferred_element_type=jnp.float32)
        u = jnp.dot(x, wu_ref[0], preferred_element_type=jnp.float32)
        h = (g * jax.nn.sigmoid(g)) * u
        y = jnp.dot(h, wd_ref[0], preferred_element_type=jnp.float32)
        y_ref[...] = y


def _ffn(xs, wg, wu, wd, tile_expert, x_blk, tile_valid):
    grid_spec = pltpu.PrefetchScalarGridSpec(
        num_scalar_prefetch=3,
        grid=(NT,),
        in_specs=[
            pl.BlockSpec((TM, D), lambda i, te, xb, vl: (xb[i], 0)),
            pl.BlockSpec((1, D, FF), lambda i, te, xb, vl: (te[i], 0, 0)),
            pl.BlockSpec((1, D, FF), lambda i, te, xb, vl: (te[i], 0, 0)),
            pl.BlockSpec((1, FF, D), lambda i, te, xb, vl: (te[i], 0, 0)),
        ],
        out_specs=pl.BlockSpec((TM, D), lambda i, te, xb, vl: (xb[i], 0)),
    )
    return pl.pallas_call(
        _ffn_body,
        grid_spec=grid_spec,
        out_shape=jax.ShapeDtypeStruct((R_PAD, D), jnp.float32),
    )(tile_expert, x_blk, tile_valid, xs, wg, wu, wd)


def _sc_combine(y, d0, d1, w0b, w1b):
    mesh = plsc.VectorSubcoreMesh(core_axis_name="c", subcore_axis_name="s")

    @functools.partial(
        pl.kernel,
        out_type=jax.ShapeDtypeStruct((T, D), jnp.float32),
        mesh=mesh,
        scratch_types=[
            pltpu.VMEM((TPW,), jnp.int32),
            pltpu.VMEM((TPW,), jnp.int32),
            pltpu.VMEM((TPW, 16), jnp.float32),
            pltpu.VMEM((TPW, 16), jnp.float32),
            pltpu.VMEM((CT, D), jnp.float32),
            pltpu.VMEM((CT, D), jnp.float32),
            pltpu.SemaphoreType.DMA,
            pltpu.SemaphoreType.DMA,
        ],
    )
    def k(y_hbm, d0_hbm, d1_hbm, w0_hbm, w1_hbm, out_hbm,
          d0_v, d1_v, w0_v, w1_v, r0_v, r1_v, s0, s1):
        wid = lax.axis_index("s") * 2 + lax.axis_index("c")
        base = wid * TPW
        pltpu.sync_copy(d0_hbm.at[pl.ds(base, TPW)], d0_v)
        pltpu.sync_copy(d1_hbm.at[pl.ds(base, TPW)], d1_v)
        pltpu.sync_copy(w0_hbm.at[pl.ds(base, TPW)], w0_v)
        pltpu.sync_copy(w1_hbm.at[pl.ds(base, TPW)], w1_v)
        for c in range(TPW // CT):
            i0 = d0_v.at[pl.ds(c * CT, CT)]
            i1 = d1_v.at[pl.ds(c * CT, CT)]
            cp0 = pltpu.async_copy(y_hbm.at[i0], r0_v, s0)
            cp1 = pltpu.async_copy(y_hbm.at[i1], r1_v, s1)
            cp0.wait()
            cp1.wait()

            def add_row(i, _):
                b0 = w0_v[c * CT + i, :]
                b1 = w1_v[c * CT + i, :]

                def add_vec(j, _):
                    r0_v[i, pl.ds(j * 16, 16)] = (
                        b0 * r0_v[i, pl.ds(j * 16, 16)]
                        + b1 * r1_v[i, pl.ds(j * 16, 16)])
                    return 0
                lax.fori_loop(0, D // 16, add_vec, 0)
                return 0

            lax.fori_loop(0, CT, add_row, 0)
            pltpu.sync_copy(r0_v, out_hbm.at[pl.ds(base + c * CT, CT)])

    return k(y, d0, d1, w0b, w1b)


def kernel(hidden_states, router_weight, w_gate, w_up, w_down):
    x = hidden_states
    i32 = jnp.int32

    e0, e1, w0, w1, r0, r1, cnt = _router(x, router_weight)
    sizes = cnt[:E]

    nt = (sizes + TM - 1) // TM
    starts = (jnp.cumsum(nt) - nt) * TM
    dest0 = (starts[e0] + r0).astype(i32)
    dest1 = (starts[e1] + r1).astype(i32)

    nt_cum = jnp.cumsum(nt)
    total_tiles = nt_cum[E - 1]
    t_ar = jnp.arange(NT, dtype=i32)
    te = jnp.searchsorted(nt_cum, t_ar, side="right").astype(i32)
    last_e = te[jnp.maximum(total_tiles - 1, 0)]
    tile_valid = (t_ar < total_tiles).astype(i32)
    tile_expert = jnp.where(t_ar < total_tiles, te, last_e).astype(i32)
    x_blk = jnp.where(t_ar < total_tiles, t_ar, total_tiles - 1).astype(i32)

    xs = _sc_dispatch(x, dest0, dest1)
    y = _ffn(xs, w_gate, w_up, w_down, tile_expert, x_blk, tile_valid)
    w0b = jnp.broadcast_to(w0[:, None], (T, 16))
    w1b = jnp.broadcast_to(w1[:, None], (T, 16))
    return _sc_combine(y, dest0, dest1, w0b, w1b)


# speedup vs baseline: 5.5481x; 1.1671x over previous
import functools

import jax
import jax.numpy as jnp
from jax import lax
from jax.experimental import pallas as pl
from jax.experimental.pallas import tpu as pltpu
from jax.experimental.pallas import tpu_sc as plsc

E = 64
K = 2
D = 2048
FF = 768
T = 2048
TB = 128
TM = 128
NT = T * K // TM + E
R_PAD = NT * TM

NW = 32
RPW = R_PAD // NW
CH = 48
NCH = R_PAD // CH
NCPW = NCH // NW
TPW = T // NW
CT = 16


def _router_body(x_ref, rw_ref, e0_ref, e1_ref, w0_ref, w1_ref,
                 r0_ref, r1_ref, cnt_ref, carry_ref):
    i = pl.program_id(0)

    @pl.when(i == 0)
    def _init():
        carry_ref[...] = jnp.zeros_like(carry_ref)

    logits = jnp.dot(x_ref[...], rw_ref[...],
                     preferred_element_type=jnp.float32)
    lane = lax.broadcasted_iota(jnp.int32, logits.shape, 1)
    m0 = jnp.max(logits, axis=1, keepdims=True)
    c0 = jnp.min(jnp.where(logits == m0, lane, E), axis=1, keepdims=True)
    oh0 = lane == c0
    masked = jnp.where(oh0, -jnp.inf, logits)
    m1 = jnp.max(masked, axis=1, keepdims=True)
    c1 = jnp.min(jnp.where(masked == m1, lane, E), axis=1, keepdims=True)
    oh1 = lane == c1
    w0 = 1.0 / (1.0 + jnp.exp(m1 - m0))
    w1 = 1.0 - w0

    oh = jnp.where(oh0 | oh1, 1.0, 0.0)
    row = lax.broadcasted_iota(jnp.int32, (TB, TB), 0)
    col = lax.broadcasted_iota(jnp.int32, (TB, TB), 1)
    tri = jnp.where(col < row, 1.0, 0.0)
    cb = jnp.dot(tri, oh, preferred_element_type=jnp.float32)
    rank = cb + carry_ref[0:1, :]
    r0 = jnp.sum(jnp.where(oh0, rank, 0.0), axis=1)
    r1 = jnp.sum(jnp.where(oh1, rank, 0.0), axis=1)
    carry_ref[...] = carry_ref[...] + jnp.sum(oh, axis=0, keepdims=True)
    cnt_ref[...] = carry_ref[...].astype(jnp.int32)

    e0_ref[0, 0, :] = c0[:, 0]
    e1_ref[0, 0, :] = c1[:, 0]
    w0_ref[0, 0, :] = w0[:, 0]
    w1_ref[0, 0, :] = w1[:, 0]
    r0_ref[0, 0, :] = r0.astype(jnp.int32)
    r1_ref[0, 0, :] = r1.astype(jnp.int32)


def _router(x, rw):
    nblk = T // TB
    i32 = jnp.int32
    f32 = jnp.float32
    outs = pl.pallas_call(
        _router_body,
        grid=(nblk,),
        in_specs=[
            pl.BlockSpec((TB, D), lambda i: (i, 0)),
            pl.BlockSpec((D, E), lambda i: (0, 0)),
        ],
        out_specs=[
            pl.BlockSpec((1, 1, TB), lambda i: (i, 0, 0)),
            pl.BlockSpec((1, 1, TB), lambda i: (i, 0, 0)),
            pl.BlockSpec((1, 1, TB), lambda i: (i, 0, 0)),
            pl.BlockSpec((1, 1, TB), lambda i: (i, 0, 0)),
            pl.BlockSpec((1, 1, TB), lambda i: (i, 0, 0)),
            pl.BlockSpec((1, 1, TB), lambda i: (i, 0, 0)),
            pl.BlockSpec((8, E), lambda i: (0, 0)),
        ],
        out_shape=[
            jax.ShapeDtypeStruct((nblk, 1, TB), i32),
            jax.ShapeDtypeStruct((nblk, 1, TB), i32),
            jax.ShapeDtypeStruct((nblk, 1, TB), f32),
            jax.ShapeDtypeStruct((nblk, 1, TB), f32),
            jax.ShapeDtypeStruct((nblk, 1, TB), i32),
            jax.ShapeDtypeStruct((nblk, 1, TB), i32),
            jax.ShapeDtypeStruct((8, E), i32),
        ],
        scratch_shapes=[pltpu.VMEM((8, E), f32)],
    )(x, rw)
    e0, e1, w0, w1, r0, r1, cnt = outs
    flat = lambda a: a.reshape(T)
    return (flat(e0), flat(e1), flat(w0), flat(w1), flat(r0), flat(r1),
            cnt)


def _index_body(cnt_ref, e0_ref, e1_ref, r0_ref, r1_ref,
                d0_ref, d1_ref, te_ref, xb_ref, vl_ref):
    f32 = jnp.float32
    i32 = jnp.int32
    sizes = cnt_ref[0:1, :].astype(f32)
    nt = jnp.floor((sizes + (TM - 1)) * (1.0 / TM))
    row = lax.broadcasted_iota(i32, (E, E), 0)
    col = lax.broadcasted_iota(i32, (E, E), 1)
    m = jnp.where(row < col, 1.0, 0.0)
    starts_t = jnp.dot(nt, m, preferred_element_type=f32)
    starts = starts_t * TM
    inc = starts_t + nt
    total = jnp.sum(nt)

    e0v = e0_ref[...]
    e1v = e1_ref[...]
    d0 = jnp.zeros_like(e0v)
    d1 = jnp.zeros_like(e1v)
    for e in range(E):
        se = starts[0, e].astype(i32)
        d0 = jnp.where(e0v == e, se, d0)
        d1 = jnp.where(e1v == e, se, d1)
    d0_ref[...] = d0 + r0_ref[...]
    d1_ref[...] = d1 + r1_ref[...]

    i_ar = lax.broadcasted_iota(i32, (1, NT), 1).astype(f32)
    te = jnp.zeros((1, NT), f32)
    for e in range(E):
        te = te + jnp.where(inc[0, e] <= i_ar, 1.0, 0.0)
    last_e = jnp.sum(jnp.where(inc <= total - 1.0, 1.0, 0.0))
    valid = i_ar < total
    te_ref[...] = jnp.where(valid, te, last_e).astype(i32)
    xb_ref[...] = jnp.where(valid, i_ar, total - 1.0).astype(i32)
    vl_ref[...] = valid.astype(i32)


def _index(cnt, e0, e1, r0, r1):
    i32 = jnp.int32
    return pl.pallas_call(
        _index_body,
        out_shape=[
            jax.ShapeDtypeStruct((T,), i32),
            jax.ShapeDtypeStruct((T,), i32),
            jax.ShapeDtypeStruct((1, NT), i32),
            jax.ShapeDtypeStruct((1, NT), i32),
            jax.ShapeDtypeStruct((1, NT), i32),
        ],
    )(cnt, e0, e1, r0, r1)


DCH = 32


def _sc_dispatch(x, d0, d1):
    mesh = plsc.VectorSubcoreMesh(core_axis_name="c", subcore_axis_name="s")

    @functools.partial(
        pl.kernel,
        out_type=jax.ShapeDtypeStruct((R_PAD, D), jnp.float32),
        mesh=mesh,
        scratch_types=[
            pltpu.VMEM((DCH, D), jnp.float32),
            pltpu.VMEM((DCH,), jnp.int32),
            pltpu.VMEM((DCH,), jnp.int32),
            pltpu.SemaphoreType.DMA,
            pltpu.SemaphoreType.DMA,
        ],
    )
    def k(x_hbm, d0_hbm, d1_hbm, xs_hbm, xv, i0_v, i1_v, s0, s1):
        wid = lax.axis_index("s") * 2 + lax.axis_index("c")
        for c in range(T // NW // DCH):
            base = wid * (T // NW) + c * DCH
            pltpu.sync_copy(x_hbm.at[pl.ds(base, DCH)], xv)
            pltpu.sync_copy(d0_hbm.at[pl.ds(base, DCH)], i0_v)
            pltpu.sync_copy(d1_hbm.at[pl.ds(base, DCH)], i1_v)
            cp0 = pltpu.async_copy(xv, xs_hbm.at[i0_v], s0)
            cp1 = pltpu.async_copy(xv, xs_hbm.at[i1_v], s1)
            cp0.wait()
            cp1.wait()

    return k(x, d0, d1)


def _ffn_body(te_ref, xb_ref, vld_ref, x_ref, wg_ref, wu_ref, wd_ref, y_ref):
    i = pl.program_id(0)

    @pl.when(vld_ref[i] == 1)
    def _():
        x = x_ref[...]
        g = jnp.dot(x, wg_ref[0], preferred_element_type=jnp.float32)
        u = jnp.dot(x, wu_ref[0], preferred_element_type=jnp.float32)
        h = (g * jax.nn.sigmoid(g)) * u
        y = jnp.dot(h, wd_ref[0], preferred_element_type=jnp.float32)
        y_ref[...] = y


def _ffn(xs, wg, wu, wd, tile_expert, x_blk, tile_valid):
    grid_spec = pltpu.PrefetchScalarGridSpec(
        num_scalar_prefetch=3,
        grid=(NT,),
        in_specs=[
            pl.BlockSpec((TM, D), lambda i, te, xb, vl: (xb[i], 0)),
            pl.BlockSpec((1, D, FF), lambda i, te, xb, vl: (te[i], 0, 0)),
            pl.BlockSpec((1, D, FF), lambda i, te, xb, vl: (te[i], 0, 0)),
            pl.BlockSpec((1, FF, D), lambda i, te, xb, vl: (te[i], 0, 0)),
        ],
        out_specs=pl.BlockSpec((TM, D), lambda i, te, xb, vl: (xb[i], 0)),
    )
    return pl.pallas_call(
        _ffn_body,
        grid_spec=grid_spec,
        out_shape=jax.ShapeDtypeStruct((R_PAD, D), jnp.float32),
    )(tile_expert, x_blk, tile_valid, xs, wg, wu, wd)


def _sc_combine(y, d0, d1, w0b, w1b):
    mesh = plsc.VectorSubcoreMesh(core_axis_name="c", subcore_axis_name="s")

    @functools.partial(
        pl.kernel,
        out_type=jax.ShapeDtypeStruct((T, D), jnp.float32),
        mesh=mesh,
        scratch_types=[
            pltpu.VMEM((TPW,), jnp.int32),
            pltpu.VMEM((TPW,), jnp.int32),
            pltpu.VMEM((TPW,), jnp.float32),
            pltpu.VMEM((TPW,), jnp.float32),
            pltpu.VMEM((CT, D), jnp.float32),
            pltpu.VMEM((CT, D), jnp.float32),
            pltpu.SemaphoreType.DMA,
            pltpu.SemaphoreType.DMA,
        ],
    )
    def k(y_hbm, d0_hbm, d1_hbm, w0_hbm, w1_hbm, out_hbm,
          d0_v, d1_v, w0_v, w1_v, r0_v, r1_v, s0, s1):
        wid = lax.axis_index("s") * 2 + lax.axis_index("c")
        base = wid * TPW
        pltpu.sync_copy(d0_hbm.at[pl.ds(base, TPW)], d0_v)
        pltpu.sync_copy(d1_hbm.at[pl.ds(base, TPW)], d1_v)
        pltpu.sync_copy(w0_hbm.at[pl.ds(base, TPW)], w0_v)
        pltpu.sync_copy(w1_hbm.at[pl.ds(base, TPW)], w1_v)
        for c in range(TPW // CT):
            i0 = d0_v.at[pl.ds(c * CT, CT)]
            i1 = d1_v.at[pl.ds(c * CT, CT)]
            cp0 = pltpu.async_copy(y_hbm.at[i0], r0_v, s0)
            cp1 = pltpu.async_copy(y_hbm.at[i1], r1_v, s1)
            w0c = w0_v[pl.ds(c * CT, CT)]
            w1c = w1_v[pl.ds(c * CT, CT)]
            cp0.wait()
            cp1.wait()

            for i in range(CT):
                b0 = jnp.zeros((16,), jnp.float32) + w0c[i]
                b1 = jnp.zeros((16,), jnp.float32) + w1c[i]

                def add_vec(j, _):
                    r0_v[i, pl.ds(j * 16, 16)] = (
                        b0 * r0_v[i, pl.ds(j * 16, 16)]
                        + b1 * r1_v[i, pl.ds(j * 16, 16)])
                    return 0
                lax.fori_loop(0, D // 16, add_vec, 0)

            pltpu.sync_copy(r0_v, out_hbm.at[pl.ds(base + c * CT, CT)])

    return k(y, d0, d1, w0b, w1b)


def kernel(hidden_states, router_weight, w_gate, w_up, w_down):
    x = hidden_states

    e0, e1, w0, w1, r0, r1, cnt = _router(x, router_weight)
    dest0, dest1, te2, xb2, vl2 = _index(cnt, e0, e1, r0, r1)
    tile_expert = te2.reshape(NT)
    x_blk = xb2.reshape(NT)
    tile_valid = vl2.reshape(NT)

    xs = _sc_dispatch(x, dest0, dest1)
    y = _ffn(xs, w_gate, w_up, w_down, tile_expert, x_blk, tile_valid)
    return _sc_combine(y, dest0, dest1, w0, w1)


# speedup vs baseline: 5.6634x; 1.0208x over previous
import functools

import jax
import jax.numpy as jnp
from jax import lax
from jax.experimental import pallas as pl
from jax.experimental.pallas import tpu as pltpu
from jax.experimental.pallas import tpu_sc as plsc

E = 64
K = 2
D = 2048
FF = 768
T = 2048
TB = 128
TM = 128
NT = T * K // TM + E
R_PAD = NT * TM

NW = 32
RPW = R_PAD // NW
CH = 48
NCH = R_PAD // CH
NCPW = NCH // NW
TPW = T // NW
CT = 8


def _router_body(x_ref, rw_ref, e0_ref, e1_ref, w0_ref, w1_ref,
                 r0_ref, r1_ref, cnt_ref, carry_ref):
    i = pl.program_id(0)

    @pl.when(i == 0)
    def _init():
        carry_ref[...] = jnp.zeros_like(carry_ref)

    logits = jnp.dot(x_ref[...], rw_ref[...],
                     preferred_element_type=jnp.float32)
    lane = lax.broadcasted_iota(jnp.int32, logits.shape, 1)
    m0 = jnp.max(logits, axis=1, keepdims=True)
    c0 = jnp.min(jnp.where(logits == m0, lane, E), axis=1, keepdims=True)
    oh0 = lane == c0
    masked = jnp.where(oh0, -jnp.inf, logits)
    m1 = jnp.max(masked, axis=1, keepdims=True)
    c1 = jnp.min(jnp.where(masked == m1, lane, E), axis=1, keepdims=True)
    oh1 = lane == c1
    w0 = 1.0 / (1.0 + jnp.exp(m1 - m0))
    w1 = 1.0 - w0

    oh = jnp.where(oh0 | oh1, 1.0, 0.0)
    row = lax.broadcasted_iota(jnp.int32, (TB, TB), 0)
    col = lax.broadcasted_iota(jnp.int32, (TB, TB), 1)
    tri = jnp.where(col < row, 1.0, 0.0)
    cb = jnp.dot(tri, oh, preferred_element_type=jnp.float32)
    rank = cb + carry_ref[0:1, :]
    r0 = jnp.sum(jnp.where(oh0, rank, 0.0), axis=1)
    r1 = jnp.sum(jnp.where(oh1, rank, 0.0), axis=1)
    carry_ref[...] = carry_ref[...] + jnp.sum(oh, axis=0, keepdims=True)
    cnt_ref[...] = carry_ref[...].astype(jnp.int32)

    e0_ref[0, 0, :] = c0[:, 0]
    e1_ref[0, 0, :] = c1[:, 0]
    w0_ref[0, 0, :] = w0[:, 0]
    w1_ref[0, 0, :] = w1[:, 0]
    r0_ref[0, 0, :] = r0.astype(jnp.int32)
    r1_ref[0, 0, :] = r1.astype(jnp.int32)


def _router(x, rw):
    nblk = T // TB
    i32 = jnp.int32
    f32 = jnp.float32
    outs = pl.pallas_call(
        _router_body,
        grid=(nblk,),
        in_specs=[
            pl.BlockSpec((TB, D), lambda i: (i, 0)),
            pl.BlockSpec((D, E), lambda i: (0, 0)),
        ],
        out_specs=[
            pl.BlockSpec((1, 1, TB), lambda i: (i, 0, 0)),
            pl.BlockSpec((1, 1, TB), lambda i: (i, 0, 0)),
            pl.BlockSpec((1, 1, TB), lambda i: (i, 0, 0)),
            pl.BlockSpec((1, 1, TB), lambda i: (i, 0, 0)),
            pl.BlockSpec((1, 1, TB), lambda i: (i, 0, 0)),
            pl.BlockSpec((1, 1, TB), lambda i: (i, 0, 0)),
            pl.BlockSpec((8, E), lambda i: (0, 0)),
        ],
        out_shape=[
            jax.ShapeDtypeStruct((nblk, 1, TB), i32),
            jax.ShapeDtypeStruct((nblk, 1, TB), i32),
            jax.ShapeDtypeStruct((nblk, 1, TB), f32),
            jax.ShapeDtypeStruct((nblk, 1, TB), f32),
            jax.ShapeDtypeStruct((nblk, 1, TB), i32),
            jax.ShapeDtypeStruct((nblk, 1, TB), i32),
            jax.ShapeDtypeStruct((8, E), i32),
        ],
        scratch_shapes=[pltpu.VMEM((8, E), f32)],
    )(x, rw)
    e0, e1, w0, w1, r0, r1, cnt = outs
    flat = lambda a: a.reshape(T)
    return (flat(e0), flat(e1), flat(w0), flat(w1), flat(r0), flat(r1),
            cnt)


def _index_body(cnt_ref, e0_ref, e1_ref, r0_ref, r1_ref,
                d0_ref, d1_ref, te_ref, xb_ref, vl_ref):
    f32 = jnp.float32
    i32 = jnp.int32
    sizes = cnt_ref[0:1, :].astype(f32)
    nt = jnp.floor((sizes + (TM - 1)) * (1.0 / TM))
    row = lax.broadcasted_iota(i32, (E, E), 0)
    col = lax.broadcasted_iota(i32, (E, E), 1)
    m = jnp.where(row < col, 1.0, 0.0)
    starts_t = jnp.dot(nt, m, preferred_element_type=f32)
    starts = starts_t * TM
    inc = starts_t + nt
    total = jnp.sum(nt)

    e0v = e0_ref[...]
    e1v = e1_ref[...]
    d0 = jnp.zeros_like(e0v)
    d1 = jnp.zeros_like(e1v)
    for e in range(E):
        se = starts[0, e].astype(i32)
        d0 = jnp.where(e0v == e, se, d0)
        d1 = jnp.where(e1v == e, se, d1)
    d0_ref[...] = d0 + r0_ref[...]
    d1_ref[...] = d1 + r1_ref[...]

    i_ar = lax.broadcasted_iota(i32, (1, NT), 1).astype(f32)
    te = jnp.zeros((1, NT), f32)
    for e in range(E):
        te = te + jnp.where(inc[0, e] <= i_ar, 1.0, 0.0)
    last_e = jnp.sum(jnp.where(inc <= total - 1.0, 1.0, 0.0))
    valid = i_ar < total
    te_ref[...] = jnp.where(valid, te, last_e).astype(i32)
    xb_ref[...] = jnp.where(valid, i_ar, total - 1.0).astype(i32)
    vl_ref[...] = valid.astype(i32)


def _index(cnt, e0, e1, r0, r1):
    i32 = jnp.int32
    return pl.pallas_call(
        _index_body,
        out_shape=[
            jax.ShapeDtypeStruct((T,), i32),
            jax.ShapeDtypeStruct((T,), i32),
            jax.ShapeDtypeStruct((1, NT), i32),
            jax.ShapeDtypeStruct((1, NT), i32),
            jax.ShapeDtypeStruct((1, NT), i32),
        ],
    )(cnt, e0, e1, r0, r1)


DCH = 32


def _sc_dispatch(x, d0, d1):
    mesh = plsc.VectorSubcoreMesh(core_axis_name="c", subcore_axis_name="s")

    @functools.partial(
        pl.kernel,
        out_type=jax.ShapeDtypeStruct((R_PAD, D), jnp.float32),
        mesh=mesh,
        scratch_types=[
            pltpu.VMEM((DCH, D), jnp.float32),
            pltpu.VMEM((DCH,), jnp.int32),
            pltpu.VMEM((DCH,), jnp.int32),
            pltpu.SemaphoreType.DMA,
            pltpu.SemaphoreType.DMA,
        ],
    )
    def k(x_hbm, d0_hbm, d1_hbm, xs_hbm, xv, i0_v, i1_v, s0, s1):
        wid = lax.axis_index("s") * 2 + lax.axis_index("c")
        for c in range(T // NW // DCH):
            base = wid * (T // NW) + c * DCH
            pltpu.sync_copy(x_hbm.at[pl.ds(base, DCH)], xv)
            pltpu.sync_copy(d0_hbm.at[pl.ds(base, DCH)], i0_v)
            pltpu.sync_copy(d1_hbm.at[pl.ds(base, DCH)], i1_v)
            cp0 = pltpu.async_copy(xv, xs_hbm.at[i0_v], s0)
            cp1 = pltpu.async_copy(xv, xs_hbm.at[i1_v], s1)
            cp0.wait()
            cp1.wait()

    return k(x, d0, d1)


def _ffn_body(te_ref, xb_ref, vld_ref, x_ref, wg_ref, wu_ref, wd_ref, y_ref):
    i = pl.program_id(0)

    @pl.when(vld_ref[i] == 1)
    def _():
        x = x_ref[...]
        g = jnp.dot(x, wg_ref[0], preferred_element_type=jnp.float32)
        u = jnp.dot(x, wu_ref[0], preferred_element_type=jnp.float32)
        h = (g * jax.nn.sigmoid(g)) * u
        y = jnp.dot(h, wd_ref[0], preferred_element_type=jnp.float32)
        y_ref[...] = y


def _ffn(xs, wg, wu, wd, tile_expert, x_blk, tile_valid):
    grid_spec = pltpu.PrefetchScalarGridSpec(
        num_scalar_prefetch=3,
        grid=(NT,),
        in_specs=[
            pl.BlockSpec((TM, D), lambda i, te, xb, vl: (xb[i], 0)),
            pl.BlockSpec((1, D, FF), lambda i, te, xb, vl: (te[i], 0, 0)),
            pl.BlockSpec((1, D, FF), lambda i, te, xb, vl: (te[i], 0, 0)),
            pl.BlockSpec((1, FF, D), lambda i, te, xb, vl: (te[i], 0, 0)),
        ],
        out_specs=pl.BlockSpec((TM, D), lambda i, te, xb, vl: (xb[i], 0)),
    )
    return pl.pallas_call(
        _ffn_body,
        grid_spec=grid_spec,
        out_shape=jax.ShapeDtypeStruct((R_PAD, D), jnp.float32),
    )(tile_expert, x_blk, tile_valid, xs, wg, wu, wd)


def _sc_combine(y, d0, d1, w0b, w1b):
    mesh = plsc.VectorSubcoreMesh(core_axis_name="c", subcore_axis_name="s")

    @functools.partial(
        pl.kernel,
        out_type=jax.ShapeDtypeStruct((T, D), jnp.float32),
        mesh=mesh,
        scratch_types=[
            pltpu.VMEM((TPW,), jnp.int32),
            pltpu.VMEM((TPW,), jnp.int32),
            pltpu.VMEM((TPW,), jnp.float32),
            pltpu.VMEM((TPW,), jnp.float32),
            pltpu.VMEM((CT, D), jnp.float32),
            pltpu.VMEM((CT, D), jnp.float32),
            pltpu.VMEM((CT, D), jnp.float32),
            pltpu.VMEM((CT, D), jnp.float32),
            pltpu.SemaphoreType.DMA,
            pltpu.SemaphoreType.DMA,
            pltpu.SemaphoreType.DMA,
            pltpu.SemaphoreType.DMA,
        ],
    )
    def k(y_hbm, d0_hbm, d1_hbm, w0_hbm, w1_hbm, out_hbm,
          d0_v, d1_v, w0_v, w1_v, r0a, r1a, r0b, r1b, s0a, s1a, s0b, s1b):
        wid = lax.axis_index("s") * 2 + lax.axis_index("c")
        base = wid * TPW
        pltpu.sync_copy(d0_hbm.at[pl.ds(base, TPW)], d0_v)
        pltpu.sync_copy(d1_hbm.at[pl.ds(base, TPW)], d1_v)
        pltpu.sync_copy(w0_hbm.at[pl.ds(base, TPW)], w0_v)
        pltpu.sync_copy(w1_hbm.at[pl.ds(base, TPW)], w1_v)

        bufs = [(r0a, r1a, s0a, s1a), (r0b, r1b, s0b, s1b)]
        nch = TPW // CT

        def start(c, buf):
            rb0, rb1, sb0, sb1 = buf
            i0 = d0_v.at[pl.ds(c * CT, CT)]
            i1 = d1_v.at[pl.ds(c * CT, CT)]
            return (pltpu.async_copy(y_hbm.at[i0], rb0, sb0),
                    pltpu.async_copy(y_hbm.at[i1], rb1, sb1))

        pend = {0: start(0, bufs[0])}
        for c in range(nch):
            rb0, rb1, _, _ = bufs[c % 2]
            if c + 1 < nch:
                pend[c + 1] = start(c + 1, bufs[(c + 1) % 2])
            cp0, cp1 = pend.pop(c)
            cp0.wait()
            cp1.wait()
            w16_0 = w0_v[pl.ds((c // 2) * 16, 16)]
            w16_1 = w1_v[pl.ds((c // 2) * 16, 16)]

            for i in range(CT):
                off = (c % 2) * CT + i
                b0 = jnp.zeros((16,), jnp.float32) + w16_0[off]
                b1 = jnp.zeros((16,), jnp.float32) + w16_1[off]

                def add_vec(j, _):
                    rb0[i, pl.ds(j * 16, 16)] = (
                        b0 * rb0[i, pl.ds(j * 16, 16)]
                        + b1 * rb1[i, pl.ds(j * 16, 16)])
                    return 0
                lax.fori_loop(0, D // 16, add_vec, 0)

            pltpu.sync_copy(rb0, out_hbm.at[pl.ds(base + c * CT, CT)])

    return k(y, d0, d1, w0b, w1b)


def kernel(hidden_states, router_weight, w_gate, w_up, w_down):
    x = hidden_states

    e0, e1, w0, w1, r0, r1, cnt = _router(x, router_weight)
    dest0, dest1, te2, xb2, vl2 = _index(cnt, e0, e1, r0, r1)
    tile_expert = te2.reshape(NT)
    x_blk = xb2.reshape(NT)
    tile_valid = vl2.reshape(NT)

    xs = _sc_dispatch(x, dest0, dest1)
    y = _ffn(xs, w_gate, w_up, w_down, tile_expert, x_blk, tile_valid)
    return _sc_combine(y, dest0, dest1, w0, w1)
